# Initial kernel scaffold; baseline (speedup 1.0000x reference)
import jax
import jax.numpy as jnp
from jax.experimental import pallas as pl


def kernel(data, embed, W_enc, a_enc, W_dec, a_dec, W_param, b_param, W_out1, b_out1, W_out2, b_out2):
    raise NotImplementedError("write your pallas kernel here")


# speedup vs baseline: 3906.6134x; 3906.6134x over previous
import jax
import jax.numpy as jnp
from jax.experimental import pallas as pl

NF = 512
ENC = 128
DEC = 128
ALPHA = 0.2
NEG = -1e30


def _leaky(x):
    return jnp.where(x >= 0, x, ALPHA * x)


def _elu(x):
    return jnp.where(x > 0, x, jnp.exp(jnp.minimum(x, 0.0)) - 1.0)


def _masked_attention(E, mask_dst, mask_src, Wh):
    pair = jnp.logical_and(mask_dst, mask_src)
    logits = jnp.where(pair, E, NEG)
    m = jnp.max(logits, axis=1, keepdims=True)
    p = jnp.where(pair, jnp.exp(logits - m), 0.0)
    denom = jnp.sum(p, axis=1, keepdims=True) + 1e-16
    att = p / denom
    return jnp.dot(att, Wh, preferred_element_type=jnp.float32, precision=jax.lax.Precision.HIGHEST)


def _fwd(data_ref, dataT_ref, embed_ref, W_enc_ref, a_enc_ref, W_dec_ref, a_dec_ref,
         W_param_ref, b_param_ref, W_out1_ref, b_out1_ref, W_out2_ref,
         b_out2_ref, pred_ref, kld_ref):
    h0 = embed_ref[:NF, :]
    e_row = embed_ref[NF:NF + 1, :]

    W1 = W_enc_ref[0]
    W2 = W_enc_ref[1]
    a1s = jnp.reshape(a_enc_ref[0, :ENC], (ENC, 1))
    a1d = jnp.reshape(a_enc_ref[0, ENC:], (ENC, 1))
    a2s = jnp.reshape(a_enc_ref[1, :ENC], (ENC, 1))
    a2d = jnp.reshape(a_enc_ref[1, ENC:], (ENC, 1))
    Wd = W_dec_ref[...]
    ads = jnp.reshape(a_dec_ref[0, :DEC], (DEC, 1))
    add = jnp.reshape(a_dec_ref[0, DEC:], (DEC, 1))
    W_param = W_param_ref[...]
    b_param = b_param_ref[...]

    Wh1 = jnp.dot(h0, W1, preferred_element_type=jnp.float32, precision=jax.lax.Precision.HIGHEST)
    es1 = jnp.dot(Wh1, a1s, preferred_element_type=jnp.float32, precision=jax.lax.Precision.HIGHEST)
    ed1 = jnp.dot(Wh1, a1d, preferred_element_type=jnp.float32, precision=jax.lax.Precision.HIGHEST)
    E1 = _leaky(ed1 + es1.T)

    wh_last = jnp.dot(e_row, Wd, preferred_element_type=jnp.float32, precision=jax.lax.Precision.HIGHEST)
    es_last = jnp.dot(wh_last, ads, preferred_element_type=jnp.float32, precision=jax.lax.Precision.HIGHEST)
    ed_last = jnp.dot(wh_last, add, preferred_element_type=jnp.float32, precision=jax.lax.Precision.HIGHEST)
    logit_last = _leaky(ed_last + es_last)

    preds = []
    klds = []
    for i in range(8):
        mask_c = dataT_ref[:, i:i + 1] != 0
        mask_r = data_ref[i:i + 1, :] != 0

        h1 = _elu(_masked_attention(E1, mask_c, mask_r, Wh1))

        Wh2 = jnp.dot(h1, W2, preferred_element_type=jnp.float32, precision=jax.lax.Precision.HIGHEST)
        es2 = jnp.dot(Wh2, a2s, preferred_element_type=jnp.float32, precision=jax.lax.Precision.HIGHEST)
        ed2 = jnp.dot(Wh2, a2d, preferred_element_type=jnp.float32, precision=jax.lax.Precision.HIGHEST)
        E2 = _leaky(ed2 + es2.T)
        h2 = _elu(_masked_attention(E2, mask_c, mask_r, Wh2))

        par = jnp.dot(h2, W_param, preferred_element_type=jnp.float32, precision=jax.lax.Precision.HIGHEST) + b_param
        mean = par[:, :DEC]
        sigma = par[:, DEC:]

        maskf = mask_c.astype(jnp.float32)
        cnt = jnp.sum(maskf)
        term = jnp.exp(sigma) - sigma - 1.0 + mean * mean
        kld = 0.5 * jnp.sum(term * maskf) / cnt
        klds.append(jnp.reshape(kld, (1, 1)))

        Whd = jnp.dot(mean, Wd, preferred_element_type=jnp.float32, precision=jax.lax.Precision.HIGHEST)
        esd = jnp.dot(Whd, ads, preferred_element_type=jnp.float32, precision=jax.lax.Precision.HIGHEST)
        lg = _leaky(ed_last + esd.T)
        lg = jnp.where(mask_r, lg, NEG)
        m = jnp.maximum(jnp.max(lg, axis=1, keepdims=True), logit_last)
        p = jnp.where(mask_r, jnp.exp(lg - m), 0.0)
        p_last = jnp.exp(logit_last - m)
        denom = jnp.sum(p, axis=1, keepdims=True) + p_last + 1e-16
        dec = (jnp.dot(p, Whd, preferred_element_type=jnp.float32, precision=jax.lax.Precision.HIGHEST)
               + p_last * wh_last) / denom
        preds.append(jax.nn.relu(dec))

    stacked = jnp.concatenate(preds, axis=0)
    hidden = jax.nn.relu(
        jnp.dot(stacked, W_out1_ref[...], preferred_element_type=jnp.float32, precision=jax.lax.Precision.HIGHEST)
        + b_out1_ref[...])
    pred_ref[...] = (jnp.dot(hidden, W_out2_ref[...],
                             preferred_element_type=jnp.float32, precision=jax.lax.Precision.HIGHEST)
                     + b_out2_ref[...])
    kld_ref[...] = sum(klds[1:], klds[0])


def kernel(data, embed, W_enc, a_enc, W_dec, a_dec, W_param, b_param,
           W_out1, b_out1, W_out2, b_out2):
    out = pl.pallas_call(
        _fwd,
        out_shape=(
            jax.ShapeDtypeStruct((8, 1), jnp.float32),
            jax.ShapeDtypeStruct((1, 1), jnp.float32),
        ),
    )(
        data.astype(jnp.int32),
        data.astype(jnp.int32).T,
        embed,
        W_enc.reshape(2, ENC, ENC),
        a_enc.reshape(2, 2 * ENC),
        W_dec.reshape(ENC, DEC),
        a_dec.reshape(1, 2 * DEC),
        W_param,
        b_param.reshape(1, 2 * ENC),
        W_out1,
        b_out1.reshape(1, DEC),
        W_out2,
        b_out2.reshape(1, 1),
    )
    prediction, kld = out
    return prediction, kld[0, 0]


# speedup vs baseline: 9994.0000x; 2.5582x over previous
import jax
import jax.numpy as jnp
from jax.experimental import pallas as pl

NF = 512
ENC = 128
DEC = 128
ALPHA = 0.2
NEG = -1e30


def _leaky(x):
    return jnp.where(x >= 0, x, ALPHA * x)


def _elu(x):
    return jnp.where(x > 0, x, jnp.exp(jnp.minimum(x, 0.0)) - 1.0)


def _masked_attention(E, mask_dst, mask_src, Wh):
    pair = jnp.logical_and(mask_dst, mask_src)
    logits = jnp.where(pair, E, NEG)
    m = jnp.max(logits, axis=1, keepdims=True)
    p = jnp.where(pair, jnp.exp(logits - m), 0.0)
    denom = jnp.sum(p, axis=1, keepdims=True) + 1e-16
    att = p / denom
    return jnp.dot(att, Wh, preferred_element_type=jnp.float32)


def _fwd(data_ref, dataT_ref, embed_ref, W_enc_ref, a_enc_ref, W_dec_ref, a_dec_ref,
         W_param_ref, b_param_ref, W_out1_ref, b_out1_ref, W_out2_ref,
         b_out2_ref, pred_ref, kld_ref):
    h0 = embed_ref[:NF, :]
    e_row = embed_ref[NF:NF + 1, :]

    W1 = W_enc_ref[0]
    W2 = W_enc_ref[1]
    a1s = jnp.reshape(a_enc_ref[0, :ENC], (ENC, 1))
    a1d = jnp.reshape(a_enc_ref[0, ENC:], (ENC, 1))
    a2s = jnp.reshape(a_enc_ref[1, :ENC], (ENC, 1))
    a2d = jnp.reshape(a_enc_ref[1, ENC:], (ENC, 1))
    Wd = W_dec_ref[...]
    ads = jnp.reshape(a_dec_ref[0, :DEC], (DEC, 1))
    add = jnp.reshape(a_dec_ref[0, DEC:], (DEC, 1))
    W_param = W_param_ref[...]
    b_param = b_param_ref[...]

    Wh1 = jnp.dot(h0, W1, preferred_element_type=jnp.float32)
    es1 = jnp.dot(Wh1, a1s, preferred_element_type=jnp.float32)
    ed1 = jnp.dot(Wh1, a1d, preferred_element_type=jnp.float32)
    E1 = _leaky(ed1 + es1.T)

    wh_last = jnp.dot(e_row, Wd, preferred_element_type=jnp.float32)
    es_last = jnp.dot(wh_last, ads, preferred_element_type=jnp.float32)
    ed_last = jnp.dot(wh_last, add, preferred_element_type=jnp.float32)
    logit_last = _leaky(ed_last + es_last)

    preds = []
    klds = []
    for i in range(8):
        mask_c = dataT_ref[:, i:i + 1] != 0
        mask_r = data_ref[i:i + 1, :] != 0

        h1 = _elu(_masked_attention(E1, mask_c, mask_r, Wh1))

        Wh2 = jnp.dot(h1, W2, preferred_element_type=jnp.float32)
        es2 = jnp.dot(Wh2, a2s, preferred_element_type=jnp.float32)
        ed2 = jnp.dot(Wh2, a2d, preferred_element_type=jnp.float32)
        E2 = _leaky(ed2 + es2.T)
        h2 = _elu(_masked_attention(E2, mask_c, mask_r, Wh2))

        par = jnp.dot(h2, W_param, preferred_element_type=jnp.float32) + b_param
        mean = par[:, :DEC]
        sigma = par[:, DEC:]

        maskf = mask_c.astype(jnp.float32)
        cnt = jnp.sum(maskf)
        term = jnp.exp(sigma) - sigma - 1.0 + mean * mean
        kld = 0.5 * jnp.sum(term * maskf) / cnt
        klds.append(jnp.reshape(kld, (1, 1)))

        Whd = jnp.dot(mean, Wd, preferred_element_type=jnp.float32)
        esd = jnp.dot(Whd, ads, preferred_element_type=jnp.float32)
        lg = _leaky(ed_last + esd.T)
        lg = jnp.where(mask_r, lg, NEG)
        m = jnp.maximum(jnp.max(lg, axis=1, keepdims=True), logit_last)
        p = jnp.where(mask_r, jnp.exp(lg - m), 0.0)
        p_last = jnp.exp(logit_last - m)
        denom = jnp.sum(p, axis=1, keepdims=True) + p_last + 1e-16
        dec = (jnp.dot(p, Whd, preferred_element_type=jnp.float32)
               + p_last * wh_last) / denom
        preds.append(jax.nn.relu(dec))

    stacked = jnp.concatenate(preds, axis=0)
    hidden = jax.nn.relu(
        jnp.dot(stacked, W_out1_ref[...], preferred_element_type=jnp.float32)
        + b_out1_ref[...])
    pred_ref[...] = (jnp.dot(hidden, W_out2_ref[...],
                             preferred_element_type=jnp.float32)
                     + b_out2_ref[...])
    kld_ref[...] = sum(klds[1:], klds[0])


def kernel(data, embed, W_enc, a_enc, W_dec, a_dec, W_param, b_param,
           W_out1, b_out1, W_out2, b_out2):
    out = pl.pallas_call(
        _fwd,
        out_shape=(
            jax.ShapeDtypeStruct((8, 1), jnp.float32),
            jax.ShapeDtypeStruct((1, 1), jnp.float32),
        ),
    )(
        data.astype(jnp.int32),
        data.astype(jnp.int32).T,
        embed,
        W_enc.reshape(2, ENC, ENC),
        a_enc.reshape(2, 2 * ENC),
        W_dec.reshape(ENC, DEC),
        a_dec.reshape(1, 2 * DEC),
        W_param,
        b_param.reshape(1, 2 * ENC),
        W_out1,
        b_out1.reshape(1, DEC),
        W_out2,
        b_out2.reshape(1, 1),
    )
    prediction, kld = out
    return prediction, kld[0, 0]
